# Initial kernel scaffold; baseline (speedup 1.0000x reference)
import jax
import jax.numpy as jnp
from jax.experimental import pallas as pl


def kernel(t, y, edge_index, W1, b1):
    raise NotImplementedError("write your pallas kernel here")


# speedup vs baseline: 9.8972x; 9.8972x over previous
import functools

import jax
import jax.numpy as jnp
from jax import lax
from jax.experimental import pallas as pl
from jax.experimental.pallas import tpu as pltpu
from jax.experimental.pallas import tpu_sc as plsc

B, N, F, E = 4, 10000, 256, 160000
BF = B * F
NC, NS, L = 2, 16, 16

N_PAD = 10240
TPR = 64
SPR = NS * TPR
NPASS = N_PAD // (NC * SPR)
WBLK = 2000
EPS = E // NS
ABUF = EPS + 64
G = 16
E_PAD1 = 163840
EPW1 = E_PAD1 // (NC * NS)
BN = 400

_mesh = functools.partial(
    plsc.VectorSubcoreMesh, core_axis_name="c", subcore_axis_name="s")
_sc_params = pltpu.CompilerParams(needs_layout_passes=False)


def _p1_body(dst_hbm, hist_hbm, dst_v, hist_v, sem):
    c = lax.axis_index("c")
    s = lax.axis_index("s")
    wid = s * NC + c
    pltpu.sync_copy(dst_hbm.at[pl.ds(wid * EPW1, EPW1)], dst_v)

    zi = jnp.zeros((L,), jnp.int32)
    onei = jnp.full((L,), 1, jnp.int32)
    iota = lax.iota(jnp.int32, L)
    fal = iota < 0

    def zfill(i, _):
        hist_v[pl.ds(i * L, L)] = zi
        return 0
    lax.fori_loop(0, N_PAD // L, zfill, 0)

    def hbody(i, _):
        dv = dst_v[pl.ds(i * L, L)]
        prev_eq = fal
        cntx = onei
        for d in range(1, L):
            g_lo = dv.at[jnp.maximum(iota - d, 0)].get(mode='promise_in_bounds')
            prev_eq = prev_eq | ((iota >= d) & (g_lo == dv))
            g_hi = dv.at[jnp.minimum(iota + d, L - 1)].get(
                mode='promise_in_bounds')
            cntx = cntx + jnp.where((iota < L - d) & (g_hi == dv), onei, zi)
        plsc.addupdate_scatter(hist_v, [dv], cntx, mask=~prev_eq)
        return 0
    lax.fori_loop(0, EPW1 // L, hbody, 0)
    pltpu.sync_copy(hist_v, hist_hbm.at[pl.ds(wid * N_PAD, N_PAD)])


def _p1(dst_pad):
    return pl.kernel(
        _p1_body,
        out_type=jax.ShapeDtypeStruct((NC * NS * N_PAD,), jnp.int32),
        mesh=_mesh(),
        compiler_params=_sc_params,
        scratch_types=[
            pltpu.VMEM((EPW1,), jnp.int32),
            pltpu.VMEM((N_PAD,), jnp.int32),
            pltpu.SemaphoreType.DMA,
        ],
    )(dst_pad)


def _p2_body(y_ref, w_ref, hist_ref, u_ref):
    xw = jnp.dot(y_ref[0], w_ref[...], preferred_element_type=jnp.float32)
    h = jnp.sum(hist_ref[0], axis=0).astype(jnp.float32)
    dinv = lax.rsqrt(1.0 + h)
    u_ref[...] = xw * dinv[:, None]


def _p2(y3, w1, hist3):
    return pl.pallas_call(
        _p2_body,
        grid=(N // BN, B),
        in_specs=[
            pl.BlockSpec((1, BN, F), lambda n, b: (b, n, 0)),
            pl.BlockSpec((F, F), lambda n, b: (0, 0)),
            pl.BlockSpec((1, NC * NS, BN), lambda n, b: (n, 0, 0)),
        ],
        out_specs=pl.BlockSpec((BN, F), lambda n, b: (n, b)),
        out_shape=jax.ShapeDtypeStruct((N, BF), jnp.float32),
    )(y3, w1, hist3)


def _p3_body(src_hbm, dst_hbm, u_hbm, acc_hbm,
             raw_s, raw_d, src_c, dst_c, cntb, cntsv, sidx, didx, rows, acc,
             sem, exs, exd, excnt):
    c = lax.axis_index("c")
    s = lax.axis_index("s")

    zi = jnp.zeros((L,), jnp.int32)
    zf = jnp.zeros((L,), jnp.float32)
    onei = jnp.full((L,), 1, jnp.int32)
    iota = lax.iota(jnp.int32, L)
    t_lo = s * TPR

    def prefix_suffix(mi):
        ss = mi
        rr = mi
        for d in (1, 2, 4, 8):
            gs = ss.at[jnp.maximum(iota - d, 0)].get(mode='promise_in_bounds')
            ss = ss + jnp.where(iota >= d, gs, zi)
            gr = rr.at[jnp.minimum(iota + d, L - 1)].get(
                mode='promise_in_bounds')
            rr = rr + jnp.where(iota < L - d, gr, zi)
        return ss, rr

    def proc_group(goff):
        sidx[pl.ds(0, L)] = src_c[pl.ds(goff, L)]
        didx[pl.ds(0, L)] = dst_c[pl.ds(goff, L)]
        pltpu.async_copy(u_hbm.at[sidx], rows, sem).wait()
        didx_v = didx[pl.ds(0, L)] * BF
        for e in range(G):
            bidx = didx_v[e]

            def strip(st, _):
                iv = bidx + st * L + iota
                plsc.addupdate_scatter(acc, [iv], rows[e, pl.ds(st * L, L)])
                return 0
            lax.fori_loop(0, BF // L, strip, 0)

    def pass_body(p, _):
        sc_base = (c * NPASS + p) * SPR

        def zacc(z, _):
            acc[pl.ds(z * L, L)] = zf
            return 0
        lax.fori_loop(0, (TPR + 1) * BF // L, zacc, 0)

        def awin(w, cv):
            pltpu.sync_copy(src_hbm.at[pl.ds(s * EPS + w * WBLK, WBLK)], raw_s)
            pltpu.sync_copy(dst_hbm.at[pl.ds(s * EPS + w * WBLK, WBLK)], raw_d)

            def abody(i, cv):
                sv = raw_s[pl.ds(i * L, L)]
                dv = raw_d[pl.ds(i * L, L)]
                m = (dv >= sc_base) & (dv < sc_base + SPR)
                mi = jnp.where(m, onei, zi)
                ss, rr = prefix_suffix(mi)
                pos = cv + ss - 1
                plsc.store_scatter(src_c, [pos], sv, mask=m)
                plsc.store_scatter(dst_c, [pos], dv - sc_base, mask=m)
                return cv + (ss + rr - mi)
            return lax.fori_loop(0, WBLK // L, abody, cv)
        cntv = lax.fori_loop(0, EPS // WBLK, awin, zi)

        cntb[pl.ds(0, L)] = cntv
        pltpu.sync_copy(cntb, excnt.at[pl.ds(s * L, L)])
        cnt = cntv[0]
        nblk = (cnt + WBLK - 1) // WBLK

        def acopy(b2, _):
            pltpu.sync_copy(src_c.at[pl.ds(b2 * WBLK, WBLK)],
                            exs.at[pl.ds(s * ABUF + b2 * WBLK, WBLK)])
            pltpu.sync_copy(dst_c.at[pl.ds(b2 * WBLK, WBLK)],
                            exd.at[pl.ds(s * ABUF + b2 * WBLK, WBLK)])
            return 0
        lax.fori_loop(0, nblk, acopy, 0)
        plsc.subcore_barrier()

        pltpu.sync_copy(excnt, cntsv)

        def jloop(j, fv):
            cj = cntsv[pl.ds(j * L, L)][0]
            nbj = (cj + WBLK - 1) // WBLK

            def bloop(b2, fv):
                pltpu.sync_copy(exs.at[pl.ds(j * ABUF + b2 * WBLK, WBLK)],
                                raw_s)
                pltpu.sync_copy(exd.at[pl.ds(j * ABUF + b2 * WBLK, WBLK)],
                                raw_d)
                lim = jnp.minimum(cj - b2 * WBLK, WBLK)

                def fbody(i, fv):
                    sv = raw_s[pl.ds(i * L, L)]
                    dl = raw_d[pl.ds(i * L, L)]
                    m = ((i * L + iota) < lim) & (dl >= t_lo) & \
                        (dl < t_lo + TPR)
                    mi = jnp.where(m, onei, zi)
                    ss, rr = prefix_suffix(mi)
                    pos = fv + ss - 1
                    plsc.store_scatter(src_c, [pos], sv, mask=m)
                    plsc.store_scatter(dst_c, [pos], dl - t_lo, mask=m)
                    return fv + (ss + rr - mi)
                fv = lax.fori_loop(0, (lim + L - 1) // L, fbody, fv)

                fc = fv[0]
                ng = fc // G

                def gbody(g, _):
                    proc_group(g * G)
                    return 0
                lax.fori_loop(0, ng, gbody, 0)
                svv = src_c[pl.ds(ng * G, L)]
                dvv = dst_c[pl.ds(ng * G, L)]
                src_c[pl.ds(0, L)] = svv
                dst_c[pl.ds(0, L)] = dvv
                return fv - ng * G
            return lax.fori_loop(0, nbj, bloop, fv)
        fv = lax.fori_loop(0, NS, jloop, zi)

        fc = fv[0]
        src_c[pl.ds(fc, L)] = zi
        dst_c[pl.ds(fc, L)] = jnp.full((L,), TPR, jnp.int32)

        @pl.when(fc > 0)
        def _():
            proc_group(0)

        pltpu.sync_copy(
            acc.at[pl.ds(0, TPR * BF)],
            acc_hbm.at[pl.ds((sc_base + s * TPR) * BF, TPR * BF)])
        plsc.subcore_barrier()
        return 0
    lax.fori_loop(0, NPASS, pass_body, 0)


def _p3(src, dst, u2):
    return pl.kernel(
        _p3_body,
        out_type=jax.ShapeDtypeStruct((N_PAD * BF,), jnp.float32),
        mesh=_mesh(),
        compiler_params=_sc_params,
        scratch_types=[
            pltpu.VMEM((WBLK,), jnp.int32),
            pltpu.VMEM((WBLK,), jnp.int32),
            pltpu.VMEM((ABUF,), jnp.int32),
            pltpu.VMEM((ABUF,), jnp.int32),
            pltpu.VMEM((L,), jnp.int32),
            pltpu.VMEM((NS * L,), jnp.int32),
            pltpu.VMEM((G,), jnp.int32),
            pltpu.VMEM((G,), jnp.int32),
            pltpu.VMEM((G, BF), jnp.float32),
            pltpu.VMEM(((TPR + 1) * BF,), jnp.float32),
            pltpu.SemaphoreType.DMA,
            pltpu.VMEM_SHARED((NS * ABUF,), jnp.int32),
            pltpu.VMEM_SHARED((NS * ABUF,), jnp.int32),
            pltpu.VMEM_SHARED((NS * L,), jnp.int32),
        ],
    )(src, dst, u2)


def _p4_body(acc_ref, u_ref, hist_ref, b1_ref, o_ref):
    h = jnp.sum(hist_ref[0], axis=0).astype(jnp.float32)
    dinv = lax.rsqrt(1.0 + h)
    o_ref[0] = dinv[:, None] * (acc_ref[...] + u_ref[...]) + b1_ref[0]


def _p4(acc2, u2, hist3, b1):
    return pl.pallas_call(
        _p4_body,
        grid=(N // BN, B),
        in_specs=[
            pl.BlockSpec((BN, F), lambda n, b: (n, b)),
            pl.BlockSpec((BN, F), lambda n, b: (n, b)),
            pl.BlockSpec((1, NC * NS, BN), lambda n, b: (n, 0, 0)),
            pl.BlockSpec((1, F), lambda n, b: (0, 0)),
        ],
        out_specs=pl.BlockSpec((1, BN, F), lambda n, b: (b, n, 0)),
        out_shape=jax.ShapeDtypeStruct((B, N, F), jnp.float32),
    )(acc2, u2, hist3, b1)


def kernel(t, y, edge_index, W1, b1):
    del t
    src = edge_index[0]
    dst = edge_index[1]
    dst_pad = jnp.concatenate(
        [dst, jnp.full((E_PAD1 - E,), N, jnp.int32)])
    hist = _p1(dst_pad)
    hist3 = hist.reshape(NC * NS, N_PAD)[:, :N]\
        .reshape(NC * NS, N // BN, BN).transpose(1, 0, 2)
    y3 = y.reshape(B, N, F)
    u2 = _p2(y3, W1, hist3)
    acc2 = _p3(src, dst, u2).reshape(N_PAD, BF)
    o3 = _p4(acc2[:N], u2, hist3, b1.reshape(1, F))
    return o3.reshape(B * N, F)


# speedup vs baseline: 11.2876x; 1.1405x over previous
import functools

import jax
import jax.numpy as jnp
from jax import lax
from jax.experimental import pallas as pl
from jax.experimental.pallas import tpu as pltpu
from jax.experimental.pallas import tpu_sc as plsc

B, N, F, E = 4, 10000, 256, 160000
BF = B * F
NC, NS, L = 2, 16, 16

N_PAD = 10240
TPR = 64
SPR = NS * TPR
NPASS = N_PAD // (NC * SPR)
WBLK = 2000
EPS = E // NS
ABUF = EPS + 64
G = 16
E_PAD1 = 163840
EPW1 = E_PAD1 // (NC * NS)
BN = 400

_mesh = functools.partial(
    plsc.VectorSubcoreMesh, core_axis_name="c", subcore_axis_name="s")
_sc_params = pltpu.CompilerParams(needs_layout_passes=False)


def _p1_body(dst_hbm, hist_hbm, dst_v, hist_v, sem):
    c = lax.axis_index("c")
    s = lax.axis_index("s")
    wid = s * NC + c
    pltpu.sync_copy(dst_hbm.at[pl.ds(wid * EPW1, EPW1)], dst_v)

    zi = jnp.zeros((L,), jnp.int32)
    onei = jnp.full((L,), 1, jnp.int32)
    iota = lax.iota(jnp.int32, L)
    fal = iota < 0

    def zfill(i, _):
        hist_v[pl.ds(i * L, L)] = zi
        return 0
    lax.fori_loop(0, N_PAD // L, zfill, 0)

    def hbody(i, _):
        dv = dst_v[pl.ds(i * L, L)]
        prev_eq = fal
        cntx = onei
        for d in range(1, L):
            g_lo = dv.at[jnp.maximum(iota - d, 0)].get(mode='promise_in_bounds')
            prev_eq = prev_eq | ((iota >= d) & (g_lo == dv))
            g_hi = dv.at[jnp.minimum(iota + d, L - 1)].get(
                mode='promise_in_bounds')
            cntx = cntx + jnp.where((iota < L - d) & (g_hi == dv), onei, zi)
        plsc.addupdate_scatter(hist_v, [dv], cntx, mask=~prev_eq)
        return 0
    lax.fori_loop(0, EPW1 // L, hbody, 0)
    pltpu.sync_copy(hist_v, hist_hbm.at[pl.ds(wid * N_PAD, N_PAD)])


def _p1(dst_pad):
    return pl.kernel(
        _p1_body,
        out_type=jax.ShapeDtypeStruct((NC * NS * N_PAD,), jnp.int32),
        mesh=_mesh(),
        compiler_params=_sc_params,
        scratch_types=[
            pltpu.VMEM((EPW1,), jnp.int32),
            pltpu.VMEM((N_PAD,), jnp.int32),
            pltpu.SemaphoreType.DMA,
        ],
    )(dst_pad)


def _p2_body(y_ref, w_ref, hist_ref, u_ref):
    xw = jnp.dot(y_ref[0], w_ref[...], preferred_element_type=jnp.float32)
    h = jnp.sum(hist_ref[0], axis=0).astype(jnp.float32)
    dinv = lax.rsqrt(1.0 + h)
    u_ref[...] = xw * dinv[:, None]


def _p2(y3, w1, hist3):
    return pl.pallas_call(
        _p2_body,
        grid=(N // BN, B),
        in_specs=[
            pl.BlockSpec((1, BN, F), lambda n, b: (b, n, 0)),
            pl.BlockSpec((F, F), lambda n, b: (0, 0)),
            pl.BlockSpec((1, NC * NS, BN), lambda n, b: (n, 0, 0)),
        ],
        out_specs=pl.BlockSpec((BN, F), lambda n, b: (n, b)),
        out_shape=jax.ShapeDtypeStruct((N, BF), jnp.float32),
    )(y3, w1, hist3)


FBUF = 2176


def _p3_body(src_hbm, dst_hbm, u_hbm, acc_hbm,
             raw_pk, pk_c, fsrc, fdst, cntb, cntsv,
             sidx_a, didx_a, rows_a, sidx_b, didx_b, rows_b, acc,
             sem_a, sem_b, exs, excnt):
    c = lax.axis_index("c")
    s = lax.axis_index("s")

    zi = jnp.zeros((L,), jnp.int32)
    zf = jnp.zeros((L,), jnp.float32)
    onei = jnp.full((L,), 1, jnp.int32)
    iota = lax.iota(jnp.int32, L)
    i15 = jnp.full((L,), L - 1, jnp.int32)
    t_lo = s * TPR

    def prefix_total(mi):
        ss = mi
        for d in (1, 2, 4, 8):
            gs = ss.at[jnp.maximum(iota - d, 0)].get(mode='promise_in_bounds')
            ss = ss + jnp.where(iota >= d, gs, zi)
        tot = ss.at[i15].get(mode='promise_in_bounds')
        return ss, tot

    def stage_fire(goff, sidx, didx, rows, sem):
        sidx[pl.ds(0, L)] = fsrc[pl.ds(goff, L)]
        didx[pl.ds(0, L)] = fdst[pl.ds(goff, L)]
        return pltpu.async_copy(u_hbm.at[sidx], rows, sem)

    def wait_add(sidx, didx, rows, sem):
        pltpu.make_async_copy(u_hbm.at[sidx], rows, sem).wait()
        didx_v = didx[pl.ds(0, L)] * BF
        for e in range(G):
            bidx = didx_v[e]

            def strip8(q, _):
                for uu in range(8):
                    st = q * 8 + uu
                    iv = bidx + st * L + iota
                    plsc.addupdate_scatter(
                        acc, [iv], rows[e, pl.ds(st * L, L)])
                return 0
            lax.fori_loop(0, BF // L // 8, strip8, 0)

    def run_groups(ng):
        @pl.when(ng > 0)
        def _():
            stage_fire(0, sidx_a, didx_a, rows_a, sem_a)

        def pair(gp, _):
            g0 = 2 * gp

            @pl.when(g0 + 1 < ng)
            def _():
                stage_fire((g0 + 1) * G, sidx_b, didx_b, rows_b, sem_b)
            wait_add(sidx_a, didx_a, rows_a, sem_a)

            @pl.when(g0 + 2 < ng)
            def _():
                stage_fire((g0 + 2) * G, sidx_a, didx_a, rows_a, sem_a)

            @pl.when(g0 + 1 < ng)
            def _():
                wait_add(sidx_b, didx_b, rows_b, sem_b)
            return 0
        lax.fori_loop(0, (ng + 1) // 2, pair, 0)

    def pass_body(p, _):
        sc_base = (c * NPASS + p) * SPR

        def zacc(z, _):
            acc[pl.ds(z * L, L)] = zf
            return 0
        lax.fori_loop(0, (TPR + 1) * BF // L, zacc, 0)

        def awin(w, cv):
            pltpu.sync_copy(src_hbm.at[pl.ds(s * EPS + w * WBLK, WBLK)],
                            raw_pk.at[pl.ds(0, WBLK)])

            def abody(i, cv):
                dv = raw_pk[pl.ds(WBLK + i * L, L)]
                sv = raw_pk[pl.ds(i * L, L)]
                m = (dv >= sc_base) & (dv < sc_base + SPR)
                mi = jnp.where(m, onei, zi)
                ss, tot = prefix_total(mi)
                pos = cv + ss - 1
                plsc.store_scatter(pk_c, [pos], sv * SPR + (dv - sc_base),
                                   mask=m)
                return cv + tot
            pltpu.sync_copy(dst_hbm.at[pl.ds(s * EPS + w * WBLK, WBLK)],
                            raw_pk.at[pl.ds(WBLK, WBLK)])
            return lax.fori_loop(0, WBLK // L, abody, cv)
        cntv = lax.fori_loop(0, EPS // WBLK, awin, zi)

        cntb[pl.ds(0, L)] = cntv
        pltpu.sync_copy(cntb, excnt.at[pl.ds(s * L, L)])
        cnt = cntv[0]
        nblk = (cnt + WBLK - 1) // WBLK

        def acopy(b2, _):
            pltpu.sync_copy(pk_c.at[pl.ds(b2 * WBLK, WBLK)],
                            exs.at[pl.ds(s * ABUF + b2 * WBLK, WBLK)])
            return 0
        lax.fori_loop(0, nblk, acopy, 0)
        plsc.subcore_barrier()

        pltpu.sync_copy(excnt, cntsv)

        def jloop(j, fv):
            cj = cntsv[pl.ds(j * L, L)][0]
            nbj = (cj + WBLK - 1) // WBLK

            def bloop(b2, fv):
                pltpu.sync_copy(exs.at[pl.ds(j * ABUF + b2 * WBLK, WBLK)],
                                raw_pk.at[pl.ds(0, WBLK)])
                lim = jnp.minimum(cj - b2 * WBLK, WBLK)

                def fbody(i, fv):
                    pv = raw_pk[pl.ds(i * L, L)]
                    dl = pv & (SPR - 1)
                    m = ((i * L + iota) < lim) & (dl >= t_lo) & \
                        (dl < t_lo + TPR)
                    mi = jnp.where(m, onei, zi)
                    ss, tot = prefix_total(mi)
                    pos = fv + ss - 1
                    plsc.store_scatter(fsrc, [pos],
                                       lax.shift_right_logical(pv, 10),
                                       mask=m)
                    plsc.store_scatter(fdst, [pos], dl - t_lo, mask=m)
                    return fv + tot
                fv = lax.fori_loop(0, (lim + L - 1) // L, fbody, fv)

                fc = fv[0]
                ng = fc // G
                run_groups(ng)
                svv = fsrc[pl.ds(ng * G, L)]
                dvv = fdst[pl.ds(ng * G, L)]
                fsrc[pl.ds(0, L)] = svv
                fdst[pl.ds(0, L)] = dvv
                return fv - ng * G
            return lax.fori_loop(0, nbj, bloop, fv)
        fv = lax.fori_loop(0, NS, jloop, zi)

        fc = fv[0]
        fsrc[pl.ds(fc, L)] = zi
        fdst[pl.ds(fc, L)] = jnp.full((L,), TPR, jnp.int32)
        run_groups((fc + G - 1) // G)

        pltpu.sync_copy(
            acc.at[pl.ds(0, TPR * BF)],
            acc_hbm.at[pl.ds((sc_base + s * TPR) * BF, TPR * BF)])
        plsc.subcore_barrier()
        return 0
    lax.fori_loop(0, NPASS, pass_body, 0)


def _p3(src, dst, u2):
    return pl.kernel(
        _p3_body,
        out_type=jax.ShapeDtypeStruct((N_PAD * BF,), jnp.float32),
        mesh=_mesh(),
        compiler_params=_sc_params,
        scratch_types=[
            pltpu.VMEM((2 * WBLK,), jnp.int32),
            pltpu.VMEM((ABUF,), jnp.int32),
            pltpu.VMEM((FBUF,), jnp.int32),
            pltpu.VMEM((FBUF,), jnp.int32),
            pltpu.VMEM((L,), jnp.int32),
            pltpu.VMEM((NS * L,), jnp.int32),
            pltpu.VMEM((G,), jnp.int32),
            pltpu.VMEM((G,), jnp.int32),
            pltpu.VMEM((G, BF), jnp.float32),
            pltpu.VMEM((G,), jnp.int32),
            pltpu.VMEM((G,), jnp.int32),
            pltpu.VMEM((G, BF), jnp.float32),
            pltpu.VMEM(((TPR + 1) * BF,), jnp.float32),
            pltpu.SemaphoreType.DMA,
            pltpu.SemaphoreType.DMA,
            pltpu.VMEM_SHARED((NS * ABUF,), jnp.int32),
            pltpu.VMEM_SHARED((NS * L,), jnp.int32),
        ],
    )(src, dst, u2)


def _p4_body(acc_ref, u_ref, hist_ref, b1_ref, o_ref):
    h = jnp.sum(hist_ref[0], axis=0).astype(jnp.float32)
    dinv = lax.rsqrt(1.0 + h)
    o_ref[0] = dinv[:, None] * (acc_ref[...] + u_ref[...]) + b1_ref[0]


def _p4(acc2, u2, hist3, b1):
    return pl.pallas_call(
        _p4_body,
        grid=(N // BN, B),
        in_specs=[
            pl.BlockSpec((BN, F), lambda n, b: (n, b)),
            pl.BlockSpec((BN, F), lambda n, b: (n, b)),
            pl.BlockSpec((1, NC * NS, BN), lambda n, b: (n, 0, 0)),
            pl.BlockSpec((1, F), lambda n, b: (0, 0)),
        ],
        out_specs=pl.BlockSpec((1, BN, F), lambda n, b: (b, n, 0)),
        out_shape=jax.ShapeDtypeStruct((B, N, F), jnp.float32),
    )(acc2, u2, hist3, b1)


def kernel(t, y, edge_index, W1, b1):
    del t
    src = edge_index[0]
    dst = edge_index[1]
    dst_pad = jnp.concatenate(
        [dst, jnp.full((E_PAD1 - E,), N, jnp.int32)])
    hist = _p1(dst_pad)
    hist3 = hist.reshape(NC * NS, N_PAD)[:, :N]\
        .reshape(NC * NS, N // BN, BN).transpose(1, 0, 2)
    y3 = y.reshape(B, N, F)
    u2 = _p2(y3, W1, hist3)
    acc2 = _p3(src, dst, u2).reshape(N_PAD, BF)
    o3 = _p4(acc2[:N], u2, hist3, b1.reshape(1, F))
    return o3.reshape(B * N, F)
